# Initial kernel scaffold; baseline (speedup 1.0000x reference)
import jax
import jax.numpy as jnp
from jax.experimental import pallas as pl


def kernel(heatmap, sizemap):
    raise NotImplementedError("write your pallas kernel here")


# speedup vs baseline: 1.0012x; 1.0012x over previous
import jax
import jax.numpy as jnp
from jax.experimental import pallas as pl

LIMIT = 200
NEG = float("-inf")


def _peaks_body(h_ref, o_ref):
    x = h_ref[0, 0]
    left = jnp.concatenate([jnp.full((256, 1), NEG, jnp.float32), x[:, :-1]], axis=1)
    right = jnp.concatenate([x[:, 1:], jnp.full((256, 1), NEG, jnp.float32)], axis=1)
    hmax = jnp.maximum(jnp.maximum(left, right), x)
    up = jnp.concatenate([jnp.full((1, 256), NEG, jnp.float32), hmax[:-1, :]], axis=0)
    down = jnp.concatenate([hmax[1:, :], jnp.full((1, 256), NEG, jnp.float32)], axis=0)
    pooled = jnp.maximum(jnp.maximum(up, down), hmax)
    o_ref[...] = jnp.where(pooled == x, x, jnp.zeros_like(x)).reshape(1, 1, 256 * 256)


def kernel(heatmap, sizemap):
    B, _, H, W = heatmap.shape
    peaks = pl.pallas_call(
        _peaks_body,
        grid=(B,),
        in_specs=[pl.BlockSpec((1, 1, H, W), lambda b: (b, 0, 0, 0))],
        out_specs=pl.BlockSpec((1, 1, H * W), lambda b: (b, 0, 0)),
        out_shape=jax.ShapeDtypeStruct((B, 1, H * W), jnp.float32),
    )(heatmap).reshape(B, H * W)
    scores, idx = jax.lax.top_k(peaks, LIMIT)
    yy = (idx // W).astype(jnp.float32)
    xx = (idx % W).astype(jnp.float32)
    size_flat = sizemap.reshape(B, 2, H * W)
    w = jnp.take_along_axis(size_flat[:, 0, :], idx, axis=1)
    h = jnp.take_along_axis(size_flat[:, 1, :], idx, axis=1)
    cx = xx / W
    cy = yy / H
    boxes = jnp.stack([cx - w / 2.0, cy - h / 2.0, cx + w / 2.0, cy + h / 2.0], axis=-1)
    return jnp.concatenate([boxes, scores[..., None]], axis=-1)


# speedup vs baseline: 2.4196x; 2.4168x over previous
import dataclasses
import functools

import jax
import jax.numpy as jnp
from jax import lax
from jax.experimental import pallas as pl
from jax.experimental.pallas import tpu as pltpu
from jax.experimental.pallas import tpu_sc as plsc

LIMIT = 200
NEG = float("-inf")
CAND_MAX = 544
SORT_PAD = 208
OUT_PAD = 1008


def _peaks_body(h_ref, o_ref, t_ref):
    x = h_ref[0, 0]
    left = jnp.concatenate([jnp.full((256, 1), NEG, jnp.float32), x[:, :-1]], axis=1)
    right = jnp.concatenate([x[:, 1:], jnp.full((256, 1), NEG, jnp.float32)], axis=1)
    hmax = jnp.maximum(jnp.maximum(left, right), x)
    up = jnp.concatenate([jnp.full((1, 256), NEG, jnp.float32), hmax[:-1, :]], axis=0)
    down = jnp.concatenate([hmax[1:, :], jnp.full((1, 256), NEG, jnp.float32)], axis=0)
    pooled = jnp.maximum(jnp.maximum(up, down), hmax)
    peaks = jnp.where(pooled == x, x, jnp.zeros_like(x))
    o_ref[...] = peaks.reshape(1, 512, 128)

    pb = lax.bitcast_convert_type(peaks, jnp.int32)

    def step(i, t):
        t_try = t | lax.shift_left(jnp.int32(1), jnp.int32(29) - i)
        cnt = jnp.sum((pb >= t_try).astype(jnp.int32))
        return jnp.where(cnt >= LIMIT, t_try, t)

    t_bits = lax.fori_loop(0, 30, step, jnp.int32(0))
    t_f = lax.bitcast_convert_type(t_bits, jnp.float32)
    t_ref[...] = jnp.full((1, 1, 128), t_f, jnp.float32)


def _sc_body(pk_hbm, th_hbm, sz_hbm, out_hbm,
             pk, tv, cv, ci, rk, sv, si, riw, rih, wr, hr, ob, sem):
    wid = lax.axis_index("s") * 2 + lax.axis_index("c")
    b = wid

    @pl.when(wid < 16)
    def _():
        pltpu.sync_copy(th_hbm.at[pl.ds(b * 128, 16)], tv)
        pltpu.sync_copy(pk_hbm.at[pl.ds(b * 65536, 65536)], pk)
        t = tv[...]
        iota = lax.iota(jnp.int32, 16)

        @pl.loop(0, CAND_MAX // 16)
        def _(k):
            cv[pl.ds(k * 16, 16)] = jnp.full((16,), -1.0, jnp.float32)

        def scan_step(i, cnt):
            v = pk[pl.ds(i * 16, 16)]
            m = v >= t

            def append():
                pos = cnt + plsc.cumsum(m.astype(jnp.int32)) - 1
                mm = m & (pos < CAND_MAX)
                plsc.store_scatter(cv, [pos], v, mask=mm)
                plsc.store_scatter(ci, [pos], i * 16 + iota, mask=mm)
                return cnt + jnp.sum(m.astype(jnp.int32))

            return lax.cond(jnp.any(m), append, lambda: cnt)

        cnt = lax.fori_loop(0, 4096, scan_step, jnp.int32(0))
        cnt = jnp.minimum(cnt, jnp.int32(CAND_MAX))
        nch = (cnt + 15) >> 4

        @pl.loop(0, CAND_MAX // 16)
        def _(k):
            rk[pl.ds(k * 16, 16)] = jnp.zeros((16,), jnp.int32)

        def rank_j(j, _):
            vj = plsc.load_gather(cv, [jnp.broadcast_to(j, (16,))])

            def rank_chunk(c, _):
                vi = cv[pl.ds(c * 16, 16)]
                ivec = c * 16 + iota
                beat = (vj > vi) | ((vj == vi) & (j < ivec))
                rk[pl.ds(c * 16, 16)] += beat.astype(jnp.int32)
                return 0

            return lax.fori_loop(0, nch, rank_chunk, 0)

        lax.fori_loop(0, nch * 16, rank_j, 0)

        def place(c, _):
            r = rk[pl.ds(c * 16, 16)]
            m = r < LIMIT
            plsc.store_scatter(sv, [r], cv[pl.ds(c * 16, 16)], mask=m)
            plsc.store_scatter(si, [r], ci[pl.ds(c * 16, 16)], mask=m)
            return 0

        lax.fori_loop(0, nch, place, 0)

        @pl.loop(0, SORT_PAD // 16)
        def _(k):
            s = si[pl.ds(k * 16, 16)] & 65535
            row = lax.shift_right_logical(s, 7)
            riw[pl.ds(k * 16, 16)] = b * 1024 + row
            rih[pl.ds(k * 16, 16)] = b * 1024 + 512 + row

        c1 = pltpu.async_copy(sz_hbm.at[riw.at[pl.ds(0, 128)]], wr.at[pl.ds(0, 128)], sem)
        c2 = pltpu.async_copy(sz_hbm.at[riw.at[pl.ds(128, 80)]], wr.at[pl.ds(128, 80)], sem)
        c3 = pltpu.async_copy(sz_hbm.at[rih.at[pl.ds(0, 128)]], hr.at[pl.ds(0, 128)], sem)
        c4 = pltpu.async_copy(sz_hbm.at[rih.at[pl.ds(128, 80)]], hr.at[pl.ds(128, 80)], sem)
        c1.wait()
        c2.wait()
        c3.wait()
        c4.wait()

        @pl.loop(0, SORT_PAD // 16)
        def _(k):
            r = k * 16 + iota
            m = r < LIMIT
            s = si[pl.ds(k * 16, 16)] & 65535
            col = s & 127
            w = plsc.load_gather(wr, [r, col])
            h = plsc.load_gather(hr, [r, col])
            xx = (s & 255).astype(jnp.float32)
            yy = lax.shift_right_logical(s, 8).astype(jnp.float32)
            cx = xx * (1.0 / 256.0)
            cy = yy * (1.0 / 256.0)
            base = r * 5
            plsc.store_scatter(ob, [base], cx - w * 0.5, mask=m)
            plsc.store_scatter(ob, [base + 1], cy - h * 0.5, mask=m)
            plsc.store_scatter(ob, [base + 2], cx + w * 0.5, mask=m)
            plsc.store_scatter(ob, [base + 3], cy + h * 0.5, mask=m)
            plsc.store_scatter(ob, [base + 4], sv[pl.ds(k * 16, 16)], mask=m)

        pltpu.sync_copy(ob, out_hbm.at[pl.ds(b * OUT_PAD, OUT_PAD)])


def kernel(heatmap, sizemap):
    B, _, H, W = heatmap.shape
    peaks, thresh = pl.pallas_call(
        _peaks_body,
        grid=(B,),
        in_specs=[pl.BlockSpec((1, 1, H, W), lambda b: (b, 0, 0, 0))],
        out_specs=[
            pl.BlockSpec((1, 512, 128), lambda b: (b, 0, 0)),
            pl.BlockSpec((1, 1, 128), lambda b: (b, 0, 0)),
        ],
        out_shape=[
            jax.ShapeDtypeStruct((B, 512, 128), jnp.float32),
            jax.ShapeDtypeStruct((B, 1, 128), jnp.float32),
        ],
    )(heatmap)

    pk_flat = peaks.reshape(B * H * W)
    th_flat = thresh.reshape(B * 128)
    sz_rows = sizemap.reshape(B * 2 * 512, 128)

    mesh = plsc.VectorSubcoreMesh(core_axis_name="c", subcore_axis_name="s")
    cp = pltpu.CompilerParams()
    if "needs_layout_passes" in pltpu.CompilerParams.__dataclass_fields__:
        cp = dataclasses.replace(cp, needs_layout_passes=False)
    sc = pl.kernel(
        _sc_body,
        mesh=mesh,
        compiler_params=cp,
        out_type=jax.ShapeDtypeStruct((B * OUT_PAD,), jnp.float32),
        scratch_types=[
            pltpu.VMEM((65536,), jnp.float32),
            pltpu.VMEM((16,), jnp.float32),
            pltpu.VMEM((CAND_MAX,), jnp.float32),
            pltpu.VMEM((CAND_MAX,), jnp.int32),
            pltpu.VMEM((CAND_MAX,), jnp.int32),
            pltpu.VMEM((SORT_PAD,), jnp.float32),
            pltpu.VMEM((SORT_PAD,), jnp.int32),
            pltpu.VMEM((SORT_PAD,), jnp.int32),
            pltpu.VMEM((SORT_PAD,), jnp.int32),
            pltpu.VMEM((SORT_PAD, 128), jnp.float32),
            pltpu.VMEM((SORT_PAD, 128), jnp.float32),
            pltpu.VMEM((OUT_PAD,), jnp.float32),
            pltpu.SemaphoreType.DMA,
        ],
    )
    out = sc(pk_flat, th_flat, sz_rows)
    return out.reshape(B, OUT_PAD)[:, : LIMIT * 5].reshape(B, LIMIT, 5)


# speedup vs baseline: 3.1040x; 1.2829x over previous
import dataclasses
import functools

import jax
import jax.numpy as jnp
from jax import lax
from jax.experimental import pallas as pl
from jax.experimental.pallas import tpu as pltpu
from jax.experimental.pallas import tpu_sc as plsc

LIMIT = 200
NEG = float("-inf")
CAND_MAX = 544
SORT_PAD = 208
OUT_PAD = 1008


def _peaks_body(h_ref, o_ref, t_ref):
    x = h_ref[:, 0]
    B = x.shape[0]
    left = jnp.concatenate([jnp.full((B, 256, 1), NEG, jnp.float32), x[:, :, :-1]], axis=2)
    right = jnp.concatenate([x[:, :, 1:], jnp.full((B, 256, 1), NEG, jnp.float32)], axis=2)
    hmax = jnp.maximum(jnp.maximum(left, right), x)
    up = jnp.concatenate([jnp.full((B, 1, 256), NEG, jnp.float32), hmax[:, :-1, :]], axis=1)
    down = jnp.concatenate([hmax[:, 1:, :], jnp.full((B, 1, 256), NEG, jnp.float32)], axis=1)
    pooled = jnp.maximum(jnp.maximum(up, down), hmax)
    peaks = jnp.where(pooled == x, x, jnp.zeros_like(x))
    o_ref[...] = peaks.reshape(B, 512, 128)

    pb = lax.bitcast_convert_type(peaks, jnp.int32)

    def step(i, t):
        t_try = t | lax.shift_left(jnp.int32(1), jnp.int32(29) - i)
        cnt = jnp.sum((pb >= t_try).astype(jnp.int32), axis=(1, 2), keepdims=True)
        return jnp.where(cnt >= LIMIT, t_try, t)

    t_bits = lax.fori_loop(0, 30, step, jnp.zeros((B, 1, 1), jnp.int32))
    t_f = lax.bitcast_convert_type(t_bits, jnp.float32)
    t_ref[...] = jnp.broadcast_to(t_f, (B, 1, 128))


def _sc_body(pk_hbm, th_hbm, sz_hbm, out_hbm,
             pk, tv, cv, ci, rk, sv, si, riw, rih, wr, hr, ob, sem):
    wid = lax.axis_index("s") * 2 + lax.axis_index("c")
    b = wid

    @pl.when(wid < 16)
    def _():
        pltpu.sync_copy(th_hbm.at[pl.ds(b * 128, 16)], tv)
        pltpu.sync_copy(pk_hbm.at[pl.ds(b * 65536, 65536)], pk)
        t = tv[...]
        iota = lax.iota(jnp.int32, 16)

        @pl.loop(0, CAND_MAX // 16)
        def _(k):
            cv[pl.ds(k * 16, 16)] = jnp.full((16,), -1.0, jnp.float32)

        def scan_step(i, cnt):
            v = pk[pl.ds(i * 16, 16)]
            m = v >= t

            def append():
                pos = cnt + plsc.cumsum(m.astype(jnp.int32)) - 1
                mm = m & (pos < CAND_MAX)
                plsc.store_scatter(cv, [pos], v, mask=mm)
                plsc.store_scatter(ci, [pos], i * 16 + iota, mask=mm)
                return cnt + jnp.sum(m.astype(jnp.int32))

            return lax.cond(jnp.any(m), append, lambda: cnt)

        cnt = lax.fori_loop(0, 4096, scan_step, jnp.int32(0))
        cnt = jnp.minimum(cnt, jnp.int32(CAND_MAX))
        nch = (cnt + 15) >> 4

        @pl.loop(0, CAND_MAX // 16)
        def _(k):
            rk[pl.ds(k * 16, 16)] = jnp.zeros((16,), jnp.int32)

        def rank_j(j, _):
            vj = plsc.load_gather(cv, [jnp.broadcast_to(j, (16,))])

            def rank_chunk(c, _):
                vi = cv[pl.ds(c * 16, 16)]
                ivec = c * 16 + iota
                beat = (vj > vi) | ((vj == vi) & (j < ivec))
                rk[pl.ds(c * 16, 16)] += beat.astype(jnp.int32)
                return 0

            return lax.fori_loop(0, nch, rank_chunk, 0)

        lax.fori_loop(0, nch * 16, rank_j, 0)

        def place(c, _):
            r = rk[pl.ds(c * 16, 16)]
            m = r < LIMIT
            plsc.store_scatter(sv, [r], cv[pl.ds(c * 16, 16)], mask=m)
            plsc.store_scatter(si, [r], ci[pl.ds(c * 16, 16)], mask=m)
            return 0

        lax.fori_loop(0, nch, place, 0)

        @pl.loop(0, SORT_PAD // 16)
        def _(k):
            s = si[pl.ds(k * 16, 16)] & 65535
            row = lax.shift_right_logical(s, 7)
            riw[pl.ds(k * 16, 16)] = b * 1024 + row
            rih[pl.ds(k * 16, 16)] = b * 1024 + 512 + row

        c1 = pltpu.async_copy(sz_hbm.at[riw.at[pl.ds(0, 128)]], wr.at[pl.ds(0, 128)], sem)
        c2 = pltpu.async_copy(sz_hbm.at[riw.at[pl.ds(128, 80)]], wr.at[pl.ds(128, 80)], sem)
        c3 = pltpu.async_copy(sz_hbm.at[rih.at[pl.ds(0, 128)]], hr.at[pl.ds(0, 128)], sem)
        c4 = pltpu.async_copy(sz_hbm.at[rih.at[pl.ds(128, 80)]], hr.at[pl.ds(128, 80)], sem)
        c1.wait()
        c2.wait()
        c3.wait()
        c4.wait()

        @pl.loop(0, SORT_PAD // 16)
        def _(k):
            r = k * 16 + iota
            m = r < LIMIT
            s = si[pl.ds(k * 16, 16)] & 65535
            col = s & 127
            w = plsc.load_gather(wr, [r, col])
            h = plsc.load_gather(hr, [r, col])
            xx = (s & 255).astype(jnp.float32)
            yy = lax.shift_right_logical(s, 8).astype(jnp.float32)
            cx = xx * (1.0 / 256.0)
            cy = yy * (1.0 / 256.0)
            base = r * 5
            plsc.store_scatter(ob, [base], cx - w * 0.5, mask=m)
            plsc.store_scatter(ob, [base + 1], cy - h * 0.5, mask=m)
            plsc.store_scatter(ob, [base + 2], cx + w * 0.5, mask=m)
            plsc.store_scatter(ob, [base + 3], cy + h * 0.5, mask=m)
            plsc.store_scatter(ob, [base + 4], sv[pl.ds(k * 16, 16)], mask=m)

        pltpu.sync_copy(ob, out_hbm.at[pl.ds(b * OUT_PAD, OUT_PAD)])


def kernel(heatmap, sizemap):
    B, _, H, W = heatmap.shape
    peaks, thresh = pl.pallas_call(
        _peaks_body,
        in_specs=[pl.BlockSpec((B, 1, H, W), lambda: (0, 0, 0, 0))],
        out_specs=[
            pl.BlockSpec((B, 512, 128), lambda: (0, 0, 0)),
            pl.BlockSpec((B, 1, 128), lambda: (0, 0, 0)),
        ],
        out_shape=[
            jax.ShapeDtypeStruct((B, 512, 128), jnp.float32),
            jax.ShapeDtypeStruct((B, 1, 128), jnp.float32),
        ],
    )(heatmap)

    pk_flat = peaks.reshape(B * H * W)
    th_flat = thresh.reshape(B * 128)
    sz_rows = sizemap.reshape(B * 2 * 512, 128)

    mesh = plsc.VectorSubcoreMesh(core_axis_name="c", subcore_axis_name="s")
    cp = pltpu.CompilerParams()
    if "needs_layout_passes" in pltpu.CompilerParams.__dataclass_fields__:
        cp = dataclasses.replace(cp, needs_layout_passes=False)
    sc = pl.kernel(
        _sc_body,
        mesh=mesh,
        compiler_params=cp,
        out_type=jax.ShapeDtypeStruct((B * OUT_PAD,), jnp.float32),
        scratch_types=[
            pltpu.VMEM((65536,), jnp.float32),
            pltpu.VMEM((16,), jnp.float32),
            pltpu.VMEM((CAND_MAX,), jnp.float32),
            pltpu.VMEM((CAND_MAX,), jnp.int32),
            pltpu.VMEM((CAND_MAX,), jnp.int32),
            pltpu.VMEM((SORT_PAD,), jnp.float32),
            pltpu.VMEM((SORT_PAD,), jnp.int32),
            pltpu.VMEM((SORT_PAD,), jnp.int32),
            pltpu.VMEM((SORT_PAD,), jnp.int32),
            pltpu.VMEM((SORT_PAD, 128), jnp.float32),
            pltpu.VMEM((SORT_PAD, 128), jnp.float32),
            pltpu.VMEM((OUT_PAD,), jnp.float32),
            pltpu.SemaphoreType.DMA,
        ],
    )
    out = sc(pk_flat, th_flat, sz_rows)
    return out.reshape(B, OUT_PAD)[:, : LIMIT * 5].reshape(B, LIMIT, 5)


# speedup vs baseline: 4.6298x; 1.4916x over previous
import dataclasses
import functools

import jax
import jax.numpy as jnp
from jax import lax
from jax.experimental import pallas as pl
from jax.experimental.pallas import tpu as pltpu
from jax.experimental.pallas import tpu_sc as plsc

LIMIT = 200
NEG = float("-inf")
CAND_MAX = 544
SORT_PAD = 208
OUT_PAD = 1008


def _peaks_body(h_ref, o_ref, t_ref):
    x = h_ref[:, 0]
    B = x.shape[0]
    left = jnp.concatenate([jnp.full((B, 256, 1), NEG, jnp.float32), x[:, :, :-1]], axis=2)
    right = jnp.concatenate([x[:, :, 1:], jnp.full((B, 256, 1), NEG, jnp.float32)], axis=2)
    hmax = jnp.maximum(jnp.maximum(left, right), x)
    up = jnp.concatenate([jnp.full((B, 1, 256), NEG, jnp.float32), hmax[:, :-1, :]], axis=1)
    down = jnp.concatenate([hmax[:, 1:, :], jnp.full((B, 1, 256), NEG, jnp.float32)], axis=1)
    pooled = jnp.maximum(jnp.maximum(up, down), hmax)
    peaks = jnp.where(pooled == x, x, jnp.zeros_like(x))
    o_ref[...] = peaks.reshape(B, 512, 128)

    pb = lax.bitcast_convert_type(peaks, jnp.int32)

    def step(i, t):
        t_try = t | lax.shift_left(jnp.int32(1), jnp.int32(29) - i)
        cnt = jnp.sum((pb >= t_try).astype(jnp.int32), axis=(1, 2), keepdims=True)
        return jnp.where(cnt >= LIMIT, t_try, t)

    t_bits = lax.fori_loop(0, 30, step, jnp.zeros((B, 1, 1), jnp.int32))
    t_f = lax.bitcast_convert_type(t_bits, jnp.float32)
    t_ref[...] = jnp.broadcast_to(t_f, (B, 1, 128))


def _sc_body(pk_hbm, th_hbm, sz_hbm, out_hbm,
             pk, tv, cv, ci, cvec, mg_cv, mg_ci, rk, svl, sil, riw, rih,
             wr, hr, ob, sh_cnt, sh_cv, sh_ci, sh_rk, sem):
    s = lax.axis_index("s")
    c = lax.axis_index("c")
    b = c * 8 + lax.shift_right_logical(s, 1)
    h = s & 1
    iota = lax.iota(jnp.int32, 16)

    pltpu.sync_copy(th_hbm.at[pl.ds(b * 128, 16)], tv)
    pltpu.sync_copy(pk_hbm.at[pl.ds(b * 65536 + h * 32768, 32768)], pk)
    t = tv[...]

    @pl.loop(0, CAND_MAX // 16)
    def _(k):
        cv[pl.ds(k * 16, 16)] = jnp.full((16,), -1.0, jnp.float32)

    def scan_step(i, cnt):
        v = pk[pl.ds(i * 16, 16)]
        m = v >= t

        def append():
            pos = cnt + plsc.cumsum(m.astype(jnp.int32)) - 1
            mm = m & (pos < CAND_MAX)
            plsc.store_scatter(cv, [pos], v, mask=mm)
            plsc.store_scatter(ci, [pos], h * 32768 + i * 16 + iota, mask=mm)
            return cnt + jnp.sum(m.astype(jnp.int32))

        return lax.cond(jnp.any(m), append, lambda: cnt)

    cnt = lax.fori_loop(0, 2048, scan_step, jnp.int32(0))
    cnt = jnp.minimum(cnt, jnp.int32(CAND_MAX))

    cvec[...] = jnp.broadcast_to(cnt, (16,))
    pltpu.sync_copy(cv, sh_cv.at[pl.ds(s * CAND_MAX, CAND_MAX)])
    pltpu.sync_copy(ci, sh_ci.at[pl.ds(s * CAND_MAX, CAND_MAX)])
    pltpu.sync_copy(cvec, sh_cnt.at[pl.ds(s * 16, 16)])
    plsc.subcore_barrier()
    lo_sid = s & ~jnp.int32(1)
    pltpu.sync_copy(sh_cv.at[pl.ds(lo_sid * CAND_MAX, CAND_MAX)], mg_cv.at[pl.ds(0, CAND_MAX)])
    pltpu.sync_copy(sh_cv.at[pl.ds((lo_sid + 1) * CAND_MAX, CAND_MAX)], mg_cv.at[pl.ds(CAND_MAX, CAND_MAX)])
    pltpu.sync_copy(sh_ci.at[pl.ds(lo_sid * CAND_MAX, CAND_MAX)], mg_ci.at[pl.ds(0, CAND_MAX)])
    pltpu.sync_copy(sh_ci.at[pl.ds((lo_sid + 1) * CAND_MAX, CAND_MAX)], mg_ci.at[pl.ds(CAND_MAX, CAND_MAX)])
    pltpu.sync_copy(sh_cnt.at[pl.ds((lo_sid + 1 - h) * 16, 16)], cvec)
    cnt_p = cvec[...][0]
    cnt_lo = jnp.where(h == 0, cnt, cnt_p)
    cnt_hi = jnp.where(h == 0, cnt_p, cnt)

    @pl.loop(0, CAND_MAX // 16)
    def _(k):
        rk[pl.ds(h * CAND_MAX + k * 16, 16)] = jnp.zeros((16,), jnp.int32)

    ch0 = h * (CAND_MAX // 16)
    nch = ch0 + ((cnt + 15) >> 4)

    def rank_seg(j_base, j_cnt):
        def rank_j(jj, _):
            j = j_base + jj
            vj = plsc.load_gather(mg_cv, [jnp.broadcast_to(j, (16,))])

            def rank_chunk(cc, _):
                vi = mg_cv[pl.ds(cc * 16, 16)]
                ivec = cc * 16 + iota
                beat = (vj > vi) | ((vj == vi) & (j < ivec))
                rk[pl.ds(cc * 16, 16)] += beat.astype(jnp.int32)
                return 0

            return lax.fori_loop(ch0, nch, rank_chunk, 0)

        lax.fori_loop(0, j_cnt, rank_j, 0)

    rank_seg(jnp.int32(0), cnt_lo)
    rank_seg(jnp.int32(CAND_MAX), cnt_hi)

    pltpu.sync_copy(rk.at[pl.ds(h * CAND_MAX, CAND_MAX)], sh_rk.at[pl.ds(s * CAND_MAX, CAND_MAX)])
    plsc.subcore_barrier()
    pltpu.sync_copy(sh_rk.at[pl.ds((lo_sid + 1 - h) * CAND_MAX, CAND_MAX)], rk.at[pl.ds((1 - h) * CAND_MAX, CAND_MAX)])

    lo_r = h * 96
    n_r = 96 + h * 8

    def place_seg(c_base, seg_cnt):
        def place(k, _):
            cc = c_base + k
            v = mg_cv[pl.ds(cc * 16, 16)]
            r = rk[pl.ds(cc * 16, 16)]
            m = (v >= 0) & (r >= lo_r) & (r < lo_r + n_r)
            plsc.store_scatter(svl, [r - lo_r], v, mask=m)
            plsc.store_scatter(sil, [r - lo_r], mg_ci[pl.ds(cc * 16, 16)], mask=m)
            return 0

        lax.fori_loop(0, (seg_cnt + 15) >> 4, place, 0)

    place_seg(jnp.int32(0), cnt_lo)
    place_seg(jnp.int32(CAND_MAX // 16), cnt_hi)

    @pl.loop(0, 7)
    def _(k):
        si16 = sil[pl.ds(k * 16, 16)] & 65535
        row = lax.shift_right_logical(si16, 7)
        riw[pl.ds(k * 16, 16)] = b * 1024 + row
        rih[pl.ds(k * 16, 16)] = b * 1024 + 512 + row

    c1 = pltpu.async_copy(sz_hbm.at[riw], wr, sem)
    c2 = pltpu.async_copy(sz_hbm.at[rih], hr, sem)
    c1.wait()
    c2.wait()

    @pl.loop(0, 7)
    def _(k):
        r_out = k * 16 + iota
        m = r_out < n_r
        si16 = sil[pl.ds(k * 16, 16)] & 65535
        col = si16 & 127
        row_sel = k * 16 + iota
        w = plsc.load_gather(wr, [row_sel, col])
        hh = plsc.load_gather(hr, [row_sel, col])
        xx = (si16 & 255).astype(jnp.float32)
        yy = lax.shift_right_logical(si16, 8).astype(jnp.float32)
        cx = xx * (1.0 / 256.0)
        cy = yy * (1.0 / 256.0)
        base = r_out * 5
        plsc.store_scatter(ob, [base], cx - w * 0.5, mask=m)
        plsc.store_scatter(ob, [base + 1], cy - hh * 0.5, mask=m)
        plsc.store_scatter(ob, [base + 2], cx + w * 0.5, mask=m)
        plsc.store_scatter(ob, [base + 3], cy + hh * 0.5, mask=m)
        plsc.store_scatter(ob, [base + 4], svl[pl.ds(k * 16, 16)], mask=m)

    @pl.when(h == 0)
    def _():
        pltpu.sync_copy(ob.at[pl.ds(0, 480)], out_hbm.at[pl.ds(b * OUT_PAD, 480)])

    @pl.when(h == 1)
    def _():
        pltpu.sync_copy(ob, out_hbm.at[pl.ds(b * OUT_PAD + 480, 528)])


def kernel(heatmap, sizemap):
    B, _, H, W = heatmap.shape
    peaks, thresh = pl.pallas_call(
        _peaks_body,
        in_specs=[pl.BlockSpec((B, 1, H, W), lambda: (0, 0, 0, 0))],
        out_specs=[
            pl.BlockSpec((B, 512, 128), lambda: (0, 0, 0)),
            pl.BlockSpec((B, 1, 128), lambda: (0, 0, 0)),
        ],
        out_shape=[
            jax.ShapeDtypeStruct((B, 512, 128), jnp.float32),
            jax.ShapeDtypeStruct((B, 1, 128), jnp.float32),
        ],
    )(heatmap)

    pk_flat = peaks.reshape(B * H * W)
    th_flat = thresh.reshape(B * 128)
    sz_rows = sizemap.reshape(B * 2 * 512, 128)

    mesh = plsc.VectorSubcoreMesh(core_axis_name="c", subcore_axis_name="s")
    cp = pltpu.CompilerParams()
    if "needs_layout_passes" in pltpu.CompilerParams.__dataclass_fields__:
        cp = dataclasses.replace(cp, needs_layout_passes=False)
    sc = pl.kernel(
        _sc_body,
        mesh=mesh,
        compiler_params=cp,
        out_type=jax.ShapeDtypeStruct((B * OUT_PAD,), jnp.float32),
        scratch_types=[
            pltpu.VMEM((32768,), jnp.float32),
            pltpu.VMEM((16,), jnp.float32),
            pltpu.VMEM((CAND_MAX,), jnp.float32),
            pltpu.VMEM((CAND_MAX,), jnp.int32),
            pltpu.VMEM((16,), jnp.int32),
            pltpu.VMEM((2 * CAND_MAX,), jnp.float32),
            pltpu.VMEM((2 * CAND_MAX,), jnp.int32),
            pltpu.VMEM((2 * CAND_MAX,), jnp.int32),
            pltpu.VMEM((112,), jnp.float32),
            pltpu.VMEM((112,), jnp.int32),
            pltpu.VMEM((112,), jnp.int32),
            pltpu.VMEM((112,), jnp.int32),
            pltpu.VMEM((112, 128), jnp.float32),
            pltpu.VMEM((112, 128), jnp.float32),
            pltpu.VMEM((528,), jnp.float32),
            pltpu.VMEM_SHARED((16 * 16,), jnp.int32),
            pltpu.VMEM_SHARED((16 * CAND_MAX,), jnp.float32),
            pltpu.VMEM_SHARED((16 * CAND_MAX,), jnp.int32),
            pltpu.VMEM_SHARED((16 * CAND_MAX,), jnp.int32),
            pltpu.SemaphoreType.DMA,
        ],
    )
    out = sc(pk_flat, th_flat, sz_rows)
    return out.reshape(B, OUT_PAD)[:, : LIMIT * 5].reshape(B, LIMIT, 5)
